# baseline (speedup 1.0000x reference)
import functools

import jax
import jax.numpy as jnp
from jax import lax
from jax.experimental import pallas as pl
from jax.experimental.pallas import tpu as pltpu
from jax.experimental.pallas import tpu_sc as plsc

_LANES = 16


def _build_kernel(batch, emb_dim, emb_rows):
    info = plsc.get_sparse_core_info()
    num_workers = info.num_cores * info.num_subcores
    assert batch % num_workers == 0
    per_w = batch // num_workers
    nb = _LANES
    rounds = per_w // nb

    mesh = plsc.VectorSubcoreMesh(core_axis_name="c", subcore_axis_name="s")

    @functools.partial(
        pl.kernel,
        mesh=mesh,
        out_type=jax.ShapeDtypeStruct((emb_dim, batch), jnp.float32),
        scratch_types=[
            pltpu.VMEM((per_w,), jnp.int32),
            pltpu.VMEM((per_w,), jnp.int32),
            pltpu.VMEM((nb, emb_dim, 128), jnp.float32),
            pltpu.VMEM((emb_dim, per_w), jnp.float32),
            pltpu.SemaphoreType.DMA,
        ],
        compiler_params=pltpu.CompilerParams(needs_layout_passes=False),
    )
    def k(act_hbm, embt_hbm, out_hbm, idx_v, col_v, blk_v, cols_v, sem):
        wid = lax.axis_index("s") * info.num_cores + lax.axis_index("c")
        base = wid * per_w
        lane = lax.iota(jnp.int32, _LANES)
        pltpu.sync_copy(act_hbm.at[pl.ds(base, per_w)], idx_v)
        for g in range(per_w // _LANES):
            x = idx_v[pl.ds(g * _LANES, _LANES)]
            x = jnp.where(x == -1, 0, x)
            x = jnp.where(x == -100, 0, x)
            x = x + 1
            idx_v[pl.ds(g * _LANES, _LANES)] = x
            col_v[pl.ds(g * _LANES, _LANES)] = jnp.bitwise_and(x, 127)

        def round_body(r, _):
            ivec = idx_v[pl.ds(r * nb, _LANES)]
            copies = []
            for t in range(nb):
                c0 = pl.multiple_of((ivec[t] // 128) * 128, 128)
                for rr in range(emb_dim // 8):
                    copies.append(
                        pltpu.async_copy(
                            embt_hbm.at[pl.ds(rr * 8, 8), pl.ds(c0, 128)],
                            blk_v.at[t, pl.ds(rr * 8, 8)],
                            sem,
                        )
                    )
            for c in copies:
                c.wait()
            col16 = col_v[pl.ds(r * nb, _LANES)]
            for d in range(emb_dim):
                v = plsc.load_gather(
                    blk_v, [lane, jnp.full((_LANES,), d, jnp.int32), col16]
                )
                cols_v[d, pl.ds(r * nb, _LANES)] = v
            return _

        lax.fori_loop(0, rounds, round_body, None)
        pltpu.sync_copy(cols_v, out_hbm.at[:, pl.ds(base, per_w)])

    return k


def kernel(action_tuple, action_emb):
    if action_tuple.ndim == 1:
        idx_col = action_tuple
    else:
        idx_col = action_tuple[:, 0]
    batch = idx_col.shape[0]
    emb_rows, emb_dim = action_emb.shape
    k = _build_kernel(batch, emb_dim, emb_rows)
    out_t = k(idx_col.astype(jnp.int32), action_emb.T)
    return out_t.T
